# baseline (speedup 1.0000x reference)
import functools

import jax
import jax.numpy as jnp
from jax import lax
from jax.experimental import pallas as pl
from jax.experimental.pallas import tpu as pltpu
from jax.experimental.pallas import tpu_sc as plsc

_D = 1024
_L = 16


@functools.lru_cache(maxsize=None)
def _make_sc_kernel(N, S, CH, NC, NS):
    NW = NC * NS
    rows_per_w = S // NW
    n_chunks = rows_per_w // CH
    assert n_chunks % 2 == 0
    mesh = plsc.VectorSubcoreMesh(core_axis_name="c", subcore_axis_name="s")

    @functools.partial(
        pl.kernel,
        mesh=mesh,
        out_type=jax.ShapeDtypeStruct((N, _D), jnp.float32),
        compiler_params=pltpu.CompilerParams(
            needs_layout_passes=False,
            skip_device_barrier=True,
            disable_bounds_checks=True,
            disable_semaphore_checks=True,
        ),
        scratch_types=[
            pltpu.VMEM((rows_per_w,), jnp.int32),
            pltpu.VMEM((2, CH, _D), jnp.float32),
            pltpu.VMEM((3 * _D,), jnp.float32),
            pltpu.SemaphoreType.DMA,
            pltpu.SemaphoreType.DMA,
            pltpu.SemaphoreType.DMA,
            pltpu.SemaphoreType.DMA,
        ],
    )
    def k(x_hbm, mask_hbm, w_hbm, out_hbm, mask_v, xbuf, w_v, sx0, sx1, so0, so1):
        c = lax.axis_index("c")
        s = lax.axis_index("s")
        wid = s * NC + c
        base = wid * rows_per_w
        sx = (sx0, sx1)
        so = (so0, so1)

        pltpu.sync_copy(w_hbm, w_v)
        pltpu.sync_copy(mask_hbm.at[pl.ds(base, rows_per_w)], mask_v)

        def issue_in(i, b):
            pltpu.async_copy(
                x_hbm.at[pl.ds(base + i * CH, CH)], xbuf.at[b], sx[b]
            )

        def wait_in(b):
            pltpu.make_async_copy(x_hbm.at[pl.ds(0, CH)], xbuf.at[b], sx[b]).wait()

        def wait_out(b):
            pltpu.make_async_copy(xbuf.at[b], out_hbm.at[pl.ds(0, CH)], so[b]).wait()

        def compute_and_flush(i, b):
            wait_in(b)
            c1 = []
            c2 = []
            for ri in range(CH):
                mb = plsc.load_gather(
                    mask_v, [jnp.full((_L,), i * CH + ri, jnp.int32)]
                )
                mf = mb.astype(jnp.float32)
                c1.append(jnp.minimum(mf, 1.0))
                c2.append(jnp.maximum(mf - 1.0, 0.0))

            def col_body(j, carry):
                w0 = w_v[pl.ds(j * _L, _L)]
                w1 = w_v[pl.ds(_D + j * _L, _L)]
                w2 = w_v[pl.ds(2 * _D + j * _L, _L)]
                d10 = w1 - w0
                d21 = w2 - w1
                for ri in range(CH):
                    t = w0 + c1[ri] * d10 + c2[ri] * d21
                    plsc.addupdate(xbuf.at[b, ri, pl.ds(j * _L, _L)], t)
                return carry

            lax.fori_loop(0, _D // _L, col_body, 0)
            pltpu.async_copy(
                xbuf.at[b], out_hbm.at[pl.ds(base + i * CH, CH)], so[b]
            )

        issue_in(0, 0)

        def pair_body(g, carry):
            for b in range(2):
                i = 2 * g + b
                nb = (b + 1) % 2

                @pl.when(i + 1 < n_chunks)
                def _():
                    @pl.when(i >= 1)
                    def _():
                        wait_out(nb)

                    issue_in(i + 1, nb)

                compute_and_flush(i, b)
            return carry

        lax.fori_loop(0, n_chunks // 2, pair_body, 0)
        wait_out(0)
        wait_out(1)

    return k


def _tc_body(x_ref, c1_ref, c2_ref, w_ref, o_ref):
    w0 = w_ref[0:1, :]
    d10 = w_ref[1:2, :] - w0
    d21 = w_ref[2:3, :] - w_ref[1:2, :]
    o_ref[...] = x_ref[...] + w0 + c1_ref[...] * d10 + c2_ref[...] * d21


@functools.lru_cache(maxsize=None)
def _make_tc_kernel(N, S, RB):
    M = N - S
    assert M % RB == 0
    base_blk = S // RB

    return pl.pallas_call(
        _tc_body,
        grid=(M // RB,),
        in_specs=[
            pl.BlockSpec((RB, _D), lambda i: (base_blk + i, 0)),
            pl.BlockSpec((RB, 1), lambda i: (base_blk + i, 0)),
            pl.BlockSpec((RB, 1), lambda i: (base_blk + i, 0)),
            pl.BlockSpec((3, _D), lambda i: (0, 0)),
        ],
        out_specs=pl.BlockSpec((RB, _D), lambda i: (i, 0)),
        out_shape=jax.ShapeDtypeStruct((M, _D), jnp.float32),
    )


def kernel(x, motion_mask, motion_embed_weight):
    B, P, D = x.shape
    if motion_mask.ndim == 1:
        motion_mask = jnp.broadcast_to(motion_mask[None, :], (B, P))
    mask = motion_mask.astype(jnp.int32).reshape(-1)
    N = B * P
    S = (N * 5) // 8
    xf = x.reshape(N, D)
    info = plsc.get_sparse_core_info()

    sc_k = _make_sc_kernel(N, S, 16, info.num_cores, info.num_subcores)
    full = sc_k(xf, mask, motion_embed_weight.reshape(-1))

    mf = mask.astype(jnp.float32)[:, None]
    c1 = jnp.minimum(mf, 1.0)
    c2 = jnp.maximum(mf - 1.0, 0.0)
    tc_k = _make_tc_kernel(N, S, 512)
    tc_part = tc_k(xf, c1, c2, motion_embed_weight)

    out = lax.dynamic_update_slice(full, tc_part, (S, 0))
    return out.reshape(B, P, D)


# speedup vs baseline: 1.0783x; 1.0783x over previous
import functools

import jax
import jax.numpy as jnp
from jax import lax
from jax.experimental import pallas as pl
from jax.experimental.pallas import tpu as pltpu
from jax.experimental.pallas import tpu_sc as plsc

_D = 1024
_L = 16


@functools.lru_cache(maxsize=None)
def _make_sc_kernel(N, CH, NC, NS):
    NW = NC * NS
    rows_per_w = N // NW
    n_chunks = rows_per_w // CH
    assert n_chunks % 2 == 0
    mesh = plsc.VectorSubcoreMesh(core_axis_name="c", subcore_axis_name="s")

    @functools.partial(
        pl.kernel,
        mesh=mesh,
        out_type=jax.ShapeDtypeStruct((N, _D), jnp.float32),
        compiler_params=pltpu.CompilerParams(
            needs_layout_passes=False,
            skip_device_barrier=True,
            disable_bounds_checks=True,
            disable_semaphore_checks=True,
        ),
        scratch_types=[
            pltpu.VMEM((rows_per_w,), jnp.int32),
            pltpu.VMEM((2, CH, _D), jnp.float32),
            pltpu.VMEM((3 * _D,), jnp.float32),
            pltpu.SemaphoreType.DMA,
            pltpu.SemaphoreType.DMA,
            pltpu.SemaphoreType.DMA,
            pltpu.SemaphoreType.DMA,
            pltpu.SemaphoreType.DMA,
            pltpu.SemaphoreType.DMA,
        ],
    )
    def k(
        x_hbm, mask_hbm, w_hbm, out_hbm, mask_v, xbuf, w_v,
        sx0, sx1, so0, so1, sw, sm,
    ):
        c = lax.axis_index("c")
        s = lax.axis_index("s")
        wid = s * NC + c
        base = wid * rows_per_w
        sx = (sx0, sx1)
        so = (so0, so1)

        w_copy = pltpu.async_copy(w_hbm, w_v, sw)
        m_copy = pltpu.async_copy(mask_hbm.at[pl.ds(base, rows_per_w)], mask_v, sm)

        def issue_in(i, b):
            pltpu.async_copy(
                x_hbm.at[pl.ds(base + i * CH, CH)], xbuf.at[b], sx[b]
            )

        def wait_in(b):
            pltpu.make_async_copy(x_hbm.at[pl.ds(0, CH)], xbuf.at[b], sx[b]).wait()

        def wait_out(b):
            pltpu.make_async_copy(xbuf.at[b], out_hbm.at[pl.ds(0, CH)], so[b]).wait()

        def compute_and_flush(i, b):
            wait_in(b)
            c1 = []
            c2 = []
            for ri in range(CH):
                mb = plsc.load_gather(
                    mask_v, [jnp.full((_L,), i * CH + ri, jnp.int32)]
                )
                mf = mb.astype(jnp.float32)
                c1.append(jnp.minimum(mf, 1.0))
                c2.append(jnp.maximum(mf - 1.0, 0.0))

            UNROLL = 4

            def col_body(jo, carry):
                for ju in range(UNROLL):
                    j = jo * UNROLL + ju
                    w0 = w_v[pl.ds(j * _L, _L)]
                    w1 = w_v[pl.ds(_D + j * _L, _L)]
                    w2 = w_v[pl.ds(2 * _D + j * _L, _L)]
                    d10 = w1 - w0
                    d21 = w2 - w1
                    for ri in range(CH):
                        t = w0 + c1[ri] * d10 + c2[ri] * d21
                        plsc.addupdate(xbuf.at[b, ri, pl.ds(j * _L, _L)], t)
                return carry

            lax.fori_loop(0, _D // _L // UNROLL, col_body, 0)
            pltpu.async_copy(
                xbuf.at[b], out_hbm.at[pl.ds(base + i * CH, CH)], so[b]
            )

        issue_in(0, 0)
        w_copy.wait()
        m_copy.wait()

        def pair_body(g, carry):
            for b in range(2):
                i = 2 * g + b
                nb = (b + 1) % 2

                @pl.when(i + 1 < n_chunks)
                def _():
                    @pl.when(i >= 1)
                    def _():
                        wait_out(nb)

                    issue_in(i + 1, nb)

                compute_and_flush(i, b)
            return carry

        lax.fori_loop(0, n_chunks // 2, pair_body, 0)
        wait_out(0)
        wait_out(1)

    return k


def kernel(x, motion_mask, motion_embed_weight):
    B, P, D = x.shape
    if motion_mask.ndim == 1:
        motion_mask = jnp.broadcast_to(motion_mask[None, :], (B, P))
    mask = motion_mask.astype(jnp.int32).reshape(-1)
    N = B * P
    info = plsc.get_sparse_core_info()
    k = _make_sc_kernel(N, 16, info.num_cores, info.num_subcores)
    out = k(x.reshape(N, D), mask, motion_embed_weight.reshape(-1))
    return out.reshape(B, P, D)
